# Initial kernel scaffold; baseline (speedup 1.0000x reference)
import jax
import jax.numpy as jnp
from jax.experimental import pallas as pl


def kernel(x_u, x_v, edge_index_adv, edge_index_dif, Wl_adv, bl_adv, Wr_adv, Wl_dif, bl_dif, Wr_dif, gamma, beta):
    raise NotImplementedError("write your pallas kernel here")


# speedup vs baseline: 2.5961x; 2.5961x over previous
import jax
import jax.numpy as jnp
from jax import lax
from jax.experimental import pallas as pl
from jax.experimental.pallas import tpu as pltpu
from jax.experimental.pallas import tpu_sc as plsc

N = 10000
D = 256
DH = 128
E = 160000
NC = 2
NS = 16
L = 16
CHUNK = 128
CPT = 80
CPT2 = CPT // NC
E_PAD = NS * CPT * CHUNK
TRASH = N
ROWS = 10112
RPT = ROWS // NS
BT = 1000


def _zero_fill(zb, width):
    zero16 = jnp.zeros((L,), jnp.float32)
    for r in range(16):
        for k in range(width // L):
            zb[r, pl.ds(k * L, L)] = zero16


def _zero_acc(zb, acc, r0):
    for k in range(RPT // 16):
        pltpu.sync_copy(zb, acc.at[pl.ds(r0 + k * 16, 16)])
    rem = RPT % 16
    if rem:
        pltpu.sync_copy(zb.at[pl.ds(0, rem)], acc.at[pl.ds(r0 + RPT - rem, rem)])


def _sum_body(xs_hbm, src_hbm, dst_hbm, sum_hbm,
              sidx, didx, rows, zb, acc, sem):
    c = lax.axis_index("c")
    s = lax.axis_index("s")
    _zero_fill(zb, DH)
    r0 = s * RPT
    _zero_acc(zb, acc, r0)
    pltpu.sync_copy(src_hbm.at[c, s], sidx)
    pltpu.sync_copy(dst_hbm.at[s], didx)
    plsc.subcore_barrier()

    def chunk(j, carry):
        pltpu.async_copy(xs_hbm.at[sidx.at[j]], rows, sem).wait()
        pltpu.sync_copy(rows, acc.at[didx.at[j]], add=True)
        return carry

    lax.fori_loop(0, CPT, chunk, None)
    plsc.subcore_barrier()
    pltpu.sync_copy(acc.at[pl.ds(r0, RPT)], sum_hbm.at[c, pl.ds(r0, RPT)])


_sc_sum = pl.kernel(
    _sum_body,
    out_type=jax.ShapeDtypeStruct((NC, ROWS, DH), jnp.float32),
    mesh=plsc.VectorSubcoreMesh(core_axis_name="c", subcore_axis_name="s"),
    scratch_types=[
        pltpu.VMEM((CPT, CHUNK), jnp.int32),
        pltpu.VMEM((CPT, CHUNK), jnp.int32),
        pltpu.VMEM((CHUNK, DH), jnp.float32),
        pltpu.VMEM((16, DH), jnp.float32),
        pltpu.VMEM_SHARED((ROWS, DH), jnp.float32),
        pltpu.SemaphoreType.DMA,
    ],
)


def _cnt_body(dst_hbm, cnt_hbm, didx, ones, zb, acc, sem):
    c = lax.axis_index("c")
    s = lax.axis_index("s")
    _zero_fill(zb, DH)
    one16 = jnp.ones((L,), jnp.float32)
    for r in range(CHUNK):
        ones[r, pl.ds(0, L)] = one16
    r0 = s * RPT
    _zero_acc(zb, acc, r0)
    pltpu.sync_copy(dst_hbm.at[s, c], didx)
    plsc.subcore_barrier()

    def chunk(j, carry):
        pltpu.sync_copy(ones, acc.at[didx.at[j]], add=True)
        return carry

    lax.fori_loop(0, CPT2, chunk, None)
    plsc.subcore_barrier()
    pltpu.sync_copy(acc.at[pl.ds(r0, RPT)], cnt_hbm.at[c, pl.ds(r0, RPT)])


_sc_cnt = pl.kernel(
    _cnt_body,
    out_type=jax.ShapeDtypeStruct((NC, ROWS, DH), jnp.float32),
    mesh=plsc.VectorSubcoreMesh(core_axis_name="c", subcore_axis_name="s"),
    scratch_types=[
        pltpu.VMEM((CPT2, CHUNK), jnp.int32),
        pltpu.VMEM((CHUNK, DH), jnp.float32),
        pltpu.VMEM((16, DH), jnp.float32),
        pltpu.VMEM_SHARED((ROWS, DH), jnp.float32),
        pltpu.SemaphoreType.DMA,
    ],
)


def _tc_body(slo_ref, shi_ref, c0_ref, c1_ref, x_ref, wlo_ref, whi_ref,
             wr_ref, bl_ref, g_ref, b_ref, o_ref):
    t = jnp.dot(slo_ref[0], wlo_ref[...], preferred_element_type=jnp.float32)
    t = t + jnp.dot(shi_ref[0], whi_ref[...], preferred_element_type=jnp.float32)
    cnt = c0_ref[0][:, 0:1] + c1_ref[0][:, 0:1]
    rec = 1.0 / jnp.maximum(cnt, 1.0)
    h = (t * rec + bl_ref[...]
         + jnp.dot(x_ref[...], wr_ref[...], preferred_element_type=jnp.float32))
    mu = jnp.mean(h, axis=-1, keepdims=True)
    d = h - mu
    var = jnp.mean(d * d, axis=-1, keepdims=True)
    y = d * lax.rsqrt(var + 1e-5) * g_ref[...] + b_ref[...]
    o_ref[...] = jnp.maximum(y, 0.0)


_encode_tc = pl.pallas_call(
    _tc_body,
    grid=(N // BT,),
    in_specs=[
        pl.BlockSpec((1, BT, DH), lambda i: (0, i, 0)),
        pl.BlockSpec((1, BT, DH), lambda i: (1, i, 0)),
        pl.BlockSpec((1, BT, DH), lambda i: (0, i, 0)),
        pl.BlockSpec((1, BT, DH), lambda i: (1, i, 0)),
        pl.BlockSpec((BT, D), lambda i: (i, 0)),
        pl.BlockSpec((DH, D), lambda i: (0, 0)),
        pl.BlockSpec((DH, D), lambda i: (0, 0)),
        pl.BlockSpec((D, D), lambda i: (0, 0)),
        pl.BlockSpec((1, D), lambda i: (0, 0)),
        pl.BlockSpec((1, D), lambda i: (0, 0)),
        pl.BlockSpec((1, D), lambda i: (0, 0)),
    ],
    out_specs=pl.BlockSpec((BT, D), lambda i: (i, 0)),
    out_shape=jax.ShapeDtypeStruct((N, D), jnp.float32),
)


def _prep_edges(edge_index):
    src = edge_index[0].astype(jnp.int32)
    dst = edge_index[1].astype(jnp.int32)
    pad = E_PAD - E
    src2 = jnp.concatenate([2 * src, jnp.zeros((pad,), jnp.int32)])
    srcp = jnp.stack([src2, src2 + 1]).reshape(NC, NS, CPT, CHUNK)
    dstp = jnp.concatenate(
        [dst, jnp.full((pad,), TRASH, jnp.int32)]).reshape(NS, CPT, CHUNK)
    return srcp, dstp


def _relation(x_src, x_dst, edge_index, Wl, bl, Wr, gamma, beta):
    srcp, dstp = _prep_edges(edge_index)
    xs = x_src.reshape(2 * N, DH)
    sum3 = _sc_sum(xs, srcp, dstp)
    cnt3 = _sc_cnt(dstp.reshape(NS, NC, CPT2, CHUNK))
    return _encode_tc(
        sum3, sum3, cnt3, cnt3, x_dst,
        Wl[:, :DH].T, Wl[:, DH:].T, Wr.T,
        bl.reshape(1, D), gamma.reshape(1, D), beta.reshape(1, D))


def kernel(x_u, x_v, edge_index_adv, edge_index_dif,
           Wl_adv, bl_adv, Wr_adv, Wl_dif, bl_dif, Wr_dif,
           gamma, beta):
    h_adv = _relation(x_u, x_v, edge_index_adv, Wl_adv, bl_adv, Wr_adv,
                      gamma, beta)
    h_dif = _relation(x_v, x_u, edge_index_dif, Wl_dif, bl_dif, Wr_dif,
                      gamma, beta)
    return (h_adv, h_dif)


# speedup vs baseline: 2.9795x; 1.1477x over previous
import jax
import jax.numpy as jnp
from jax import lax
from jax.experimental import pallas as pl
from jax.experimental.pallas import tpu as pltpu
from jax.experimental.pallas import tpu_sc as plsc

N = 10000
D = 256
DH = 128
E = 160000
NC = 2
NS = 16
L = 16
CHUNK = 64
CPT = 160
EPT = CPT * CHUNK
CPT2 = CPT // NC
E_PAD = NS * CPT * CHUNK
TRASH = N
ROWS = 10112
RPT = ROWS // NS
BT = 1000


def _zero_fill(zb, width):
    zero16 = jnp.zeros((L,), jnp.float32)
    for r in range(16):
        for k in range(width // L):
            zb[r, pl.ds(k * L, L)] = zero16


def _zero_acc(zb, acc, r0):
    for k in range(RPT // 16):
        pltpu.sync_copy(zb, acc.at[pl.ds(r0 + k * 16, 16)])
    rem = RPT % 16
    if rem:
        pltpu.sync_copy(zb.at[pl.ds(0, rem)], acc.at[pl.ds(r0 + RPT - rem, rem)])


def _sum_body(xs_hbm, src_hbm, dst_hbm, sum_hbm,
              sidx, didx, rows, zb, acc, sem):
    c = lax.axis_index("c")
    s = lax.axis_index("s")
    _zero_fill(zb, DH)
    r0 = s * RPT
    _zero_acc(zb, acc, r0)
    pltpu.sync_copy(src_hbm.at[s], sidx)
    pltpu.sync_copy(dst_hbm.at[s], didx)
    cb = jnp.full((L,), c, jnp.int32)

    def fix(t, carry):
        v = sidx[pl.ds(t * L, L)]
        sidx[pl.ds(t * L, L)] = v + cb
        return carry

    lax.fori_loop(0, EPT // L, fix, None)
    plsc.subcore_barrier()

    def gref(a):
        return xs_hbm.at[sidx.at[pl.ds(a * CHUNK, CHUNK)]]

    pltpu.async_copy(gref(0), rows.at[0], sem.at[0])

    def chunk(j, carry):
        b = lax.rem(j, 2)
        bn = lax.rem(j + 1, 2)

        @pl.when(j < CPT - 1)
        def _():
            pltpu.async_copy(gref(j + 1), rows.at[bn], sem.at[bn])
        pltpu.make_async_copy(gref(j), rows.at[b], sem.at[b]).wait()
        pltpu.sync_copy(rows.at[b], acc.at[didx.at[j]], add=True)
        return carry

    lax.fori_loop(0, CPT, chunk, None)
    plsc.subcore_barrier()
    pltpu.sync_copy(acc.at[pl.ds(r0, RPT)], sum_hbm.at[c, pl.ds(r0, RPT)])


_sc_sum = pl.kernel(
    _sum_body,
    out_type=jax.ShapeDtypeStruct((NC, ROWS, DH), jnp.float32),
    mesh=plsc.VectorSubcoreMesh(core_axis_name="c", subcore_axis_name="s"),
    scratch_types=[
        pltpu.VMEM((EPT,), jnp.int32),
        pltpu.VMEM((CPT, CHUNK), jnp.int32),
        pltpu.VMEM((2, CHUNK, DH), jnp.float32),
        pltpu.VMEM((16, DH), jnp.float32),
        pltpu.VMEM_SHARED((ROWS, DH), jnp.float32),
        pltpu.SemaphoreType.DMA((2,)),
    ],
)


def _cnt_body(dst_hbm, cnt_hbm, didx, ones, zb, acc, sem):
    c = lax.axis_index("c")
    s = lax.axis_index("s")
    _zero_fill(zb, DH)
    one16 = jnp.ones((L,), jnp.float32)
    for r in range(CHUNK):
        ones[r, pl.ds(0, L)] = one16
    r0 = s * RPT
    _zero_acc(zb, acc, r0)
    pltpu.sync_copy(dst_hbm.at[s, c], didx)
    plsc.subcore_barrier()

    def chunk(j, carry):
        pltpu.sync_copy(ones, acc.at[didx.at[j]], add=True)
        return carry

    lax.fori_loop(0, CPT2, chunk, None)
    plsc.subcore_barrier()
    pltpu.sync_copy(acc.at[pl.ds(r0, RPT)], cnt_hbm.at[c, pl.ds(r0, RPT)])


_sc_cnt = pl.kernel(
    _cnt_body,
    out_type=jax.ShapeDtypeStruct((NC, ROWS, DH), jnp.float32),
    mesh=plsc.VectorSubcoreMesh(core_axis_name="c", subcore_axis_name="s"),
    scratch_types=[
        pltpu.VMEM((CPT2, CHUNK), jnp.int32),
        pltpu.VMEM((CHUNK, DH), jnp.float32),
        pltpu.VMEM((16, DH), jnp.float32),
        pltpu.VMEM_SHARED((ROWS, DH), jnp.float32),
        pltpu.SemaphoreType.DMA,
    ],
)


def _tc_body(slo_ref, shi_ref, c0_ref, c1_ref, x_ref, wlo_ref, whi_ref,
             wr_ref, bl_ref, g_ref, b_ref, o_ref):
    t = jnp.dot(slo_ref[0], wlo_ref[...], preferred_element_type=jnp.float32)
    t = t + jnp.dot(shi_ref[0], whi_ref[...], preferred_element_type=jnp.float32)
    cnt = c0_ref[0][:, 0:1] + c1_ref[0][:, 0:1]
    rec = 1.0 / jnp.maximum(cnt, 1.0)
    h = (t * rec + bl_ref[...]
         + jnp.dot(x_ref[...], wr_ref[...], preferred_element_type=jnp.float32))
    mu = jnp.mean(h, axis=-1, keepdims=True)
    d = h - mu
    var = jnp.mean(d * d, axis=-1, keepdims=True)
    y = d * lax.rsqrt(var + 1e-5) * g_ref[...] + b_ref[...]
    o_ref[...] = jnp.maximum(y, 0.0)


_encode_tc = pl.pallas_call(
    _tc_body,
    grid=(N // BT,),
    in_specs=[
        pl.BlockSpec((1, BT, DH), lambda i: (0, i, 0)),
        pl.BlockSpec((1, BT, DH), lambda i: (1, i, 0)),
        pl.BlockSpec((1, BT, DH), lambda i: (0, i, 0)),
        pl.BlockSpec((1, BT, DH), lambda i: (1, i, 0)),
        pl.BlockSpec((BT, D), lambda i: (i, 0)),
        pl.BlockSpec((DH, D), lambda i: (0, 0)),
        pl.BlockSpec((DH, D), lambda i: (0, 0)),
        pl.BlockSpec((D, D), lambda i: (0, 0)),
        pl.BlockSpec((1, D), lambda i: (0, 0)),
        pl.BlockSpec((1, D), lambda i: (0, 0)),
        pl.BlockSpec((1, D), lambda i: (0, 0)),
    ],
    out_specs=pl.BlockSpec((BT, D), lambda i: (i, 0)),
    out_shape=jax.ShapeDtypeStruct((N, D), jnp.float32),
)


def _prep_edges(edge_index):
    src = edge_index[0].astype(jnp.int32)
    dst = edge_index[1].astype(jnp.int32)
    pad = E_PAD - E
    srcp = jnp.concatenate([2 * src, jnp.zeros((pad,), jnp.int32)]
                           ).reshape(NS, EPT)
    dstp = jnp.concatenate(
        [dst, jnp.full((pad,), TRASH, jnp.int32)]).reshape(NS, CPT, CHUNK)
    return srcp, dstp


def _relation(x_src, x_dst, edge_index, Wl, bl, Wr, gamma, beta):
    srcp, dstp = _prep_edges(edge_index)
    xs = x_src.reshape(2 * N, DH)
    sum3 = _sc_sum(xs, srcp, dstp)
    cnt3 = _sc_cnt(dstp.reshape(NS, NC, CPT2, CHUNK))
    return _encode_tc(
        sum3, sum3, cnt3, cnt3, x_dst,
        Wl[:, :DH].T, Wl[:, DH:].T, Wr.T,
        bl.reshape(1, D), gamma.reshape(1, D), beta.reshape(1, D))


def kernel(x_u, x_v, edge_index_adv, edge_index_dif,
           Wl_adv, bl_adv, Wr_adv, Wl_dif, bl_dif, Wr_dif,
           gamma, beta):
    h_adv = _relation(x_u, x_v, edge_index_adv, Wl_adv, bl_adv, Wr_adv,
                      gamma, beta)
    h_dif = _relation(x_v, x_u, edge_index_dif, Wl_dif, bl_dif, Wr_dif,
                      gamma, beta)
    return (h_adv, h_dif)


# speedup vs baseline: 5.7965x; 1.9454x over previous
import jax
import jax.numpy as jnp
from jax import lax
from jax.experimental import pallas as pl
from jax.experimental.pallas import tpu as pltpu
from jax.experimental.pallas import tpu_sc as plsc

N = 10000
D = 256
DH = 128
E = 160000
NC = 2
NS = 16
L = 16
CHUNK = 80
CPT = 125
EPT = CPT * CHUNK
CHUNKC = 40
CPTC = 125
ROWS = 10112
RPT = ROWS // NS
BT = 1000


def _zero_fill(zb, width):
    zero16 = jnp.zeros((L,), jnp.float32)
    for r in range(16):
        for k in range(width // L):
            zb[r, pl.ds(k * L, L)] = zero16


def _zero_acc(zb, acc, r0):
    for k in range(RPT // 16):
        pltpu.sync_copy(zb, acc.at[pl.ds(r0 + k * 16, 16)])
    rem = RPT % 16
    if rem:
        pltpu.sync_copy(zb.at[pl.ds(0, rem)], acc.at[pl.ds(r0 + RPT - rem, rem)])


def _sum_body(xs_hbm, src_hbm, dst_hbm, sum_hbm,
              sidx, didx, rows, zb, acc, sem):
    c = lax.axis_index("c")
    s = lax.axis_index("s")
    _zero_fill(zb, DH)
    r0 = s * RPT
    _zero_acc(zb, acc, r0)
    pltpu.sync_copy(src_hbm.at[s], sidx)
    pltpu.sync_copy(dst_hbm.at[s], didx)
    cb = jnp.full((L,), c, jnp.int32)

    def fix(t, carry):
        v = sidx[pl.ds(t * L, L)]
        sidx[pl.ds(t * L, L)] = v + cb
        return carry

    lax.fori_loop(0, EPT // L, fix, None)
    plsc.subcore_barrier()

    def gref(a):
        return xs_hbm.at[sidx.at[pl.ds(a * CHUNK, CHUNK)]]

    pltpu.async_copy(gref(0), rows.at[0], sem.at[0])

    def chunk(j, carry):
        b = lax.rem(j, 2)
        bn = lax.rem(j + 1, 2)

        @pl.when(j < CPT - 1)
        def _():
            pltpu.async_copy(gref(j + 1), rows.at[bn], sem.at[bn])
        pltpu.make_async_copy(gref(j), rows.at[b], sem.at[b]).wait()
        pltpu.sync_copy(rows.at[b], acc.at[didx.at[j]], add=True)
        return carry

    lax.fori_loop(0, CPT, chunk, None)
    plsc.subcore_barrier()
    pltpu.sync_copy(acc.at[pl.ds(r0, RPT)], sum_hbm.at[c, pl.ds(r0, RPT)])


_sc_sum = pl.kernel(
    _sum_body,
    out_type=jax.ShapeDtypeStruct((NC, ROWS, DH), jnp.float32),
    mesh=plsc.VectorSubcoreMesh(core_axis_name="c", subcore_axis_name="s"),
    scratch_types=[
        pltpu.VMEM((EPT,), jnp.int32),
        pltpu.VMEM((CPT, CHUNK), jnp.int32),
        pltpu.VMEM((2, CHUNK, DH), jnp.float32),
        pltpu.VMEM((16, DH), jnp.float32),
        pltpu.VMEM_SHARED((ROWS, DH), jnp.float32),
        pltpu.SemaphoreType.DMA((2,)),
    ],
)


def _cnt_body(dst_hbm, cnt_hbm, didx, ones, zb, acc, sem):
    c = lax.axis_index("c")
    s = lax.axis_index("s")
    _zero_fill(zb, DH)
    one16 = jnp.ones((L,), jnp.float32)
    for r in range(CHUNKC):
        ones[r, pl.ds(0, L)] = one16
    r0 = s * RPT
    _zero_acc(zb, acc, r0)
    pltpu.sync_copy(dst_hbm.at[c * NS + s], didx)
    plsc.subcore_barrier()

    def chunk(j, carry):
        pltpu.sync_copy(ones, acc.at[didx.at[j]], add=True)
        return carry

    lax.fori_loop(0, CPTC, chunk, None)
    plsc.subcore_barrier()
    pltpu.sync_copy(acc.at[pl.ds(r0, RPT)], cnt_hbm.at[c, pl.ds(r0, RPT)])


_sc_cnt = pl.kernel(
    _cnt_body,
    out_type=jax.ShapeDtypeStruct((NC, ROWS, DH), jnp.float32),
    mesh=plsc.VectorSubcoreMesh(core_axis_name="c", subcore_axis_name="s"),
    scratch_types=[
        pltpu.VMEM((CPTC, CHUNKC), jnp.int32),
        pltpu.VMEM((CHUNKC, DH), jnp.float32),
        pltpu.VMEM((16, DH), jnp.float32),
        pltpu.VMEM_SHARED((ROWS, DH), jnp.float32),
        pltpu.SemaphoreType.DMA,
    ],
)


def _tc_body(slo_ref, shi_ref, c0_ref, c1_ref, x_ref, wlo_ref, whi_ref,
             wr_ref, bl_ref, g_ref, b_ref, o_ref):
    t = jnp.dot(slo_ref[0], wlo_ref[...], preferred_element_type=jnp.float32)
    t = t + jnp.dot(shi_ref[0], whi_ref[...], preferred_element_type=jnp.float32)
    cnt = c0_ref[0][:, 0:1] + c1_ref[0][:, 0:1]
    rec = 1.0 / jnp.maximum(cnt, 1.0)
    h = (t * rec + bl_ref[...]
         + jnp.dot(x_ref[...], wr_ref[...], preferred_element_type=jnp.float32))
    mu = jnp.mean(h, axis=-1, keepdims=True)
    d = h - mu
    var = jnp.mean(d * d, axis=-1, keepdims=True)
    y = d * lax.rsqrt(var + 1e-5) * g_ref[...] + b_ref[...]
    o_ref[...] = jnp.maximum(y, 0.0)


_encode_tc = pl.pallas_call(
    _tc_body,
    grid=(N // BT,),
    in_specs=[
        pl.BlockSpec((1, BT, DH), lambda i: (0, i, 0)),
        pl.BlockSpec((1, BT, DH), lambda i: (1, i, 0)),
        pl.BlockSpec((1, BT, DH), lambda i: (0, i, 0)),
        pl.BlockSpec((1, BT, DH), lambda i: (1, i, 0)),
        pl.BlockSpec((BT, D), lambda i: (i, 0)),
        pl.BlockSpec((DH, D), lambda i: (0, 0)),
        pl.BlockSpec((DH, D), lambda i: (0, 0)),
        pl.BlockSpec((D, D), lambda i: (0, 0)),
        pl.BlockSpec((1, D), lambda i: (0, 0)),
        pl.BlockSpec((1, D), lambda i: (0, 0)),
        pl.BlockSpec((1, D), lambda i: (0, 0)),
    ],
    out_specs=pl.BlockSpec((BT, D), lambda i: (i, 0)),
    out_shape=jax.ShapeDtypeStruct((N, D), jnp.float32),
)


def _prep_edges(edge_index):
    src = edge_index[0].astype(jnp.int32)
    dst = edge_index[1].astype(jnp.int32)
    srcp = (2 * src).reshape(NS, EPT)
    dstp = dst.reshape(NS, CPT, CHUNK)
    dstc = dst.reshape(NC * NS, CPTC, CHUNKC)
    return srcp, dstp, dstc


def _relation(x_src, x_dst, edge_index, Wl, bl, Wr, gamma, beta):
    srcp, dstp, dstc = _prep_edges(edge_index)
    xs = x_src.reshape(2 * N, DH)
    sum3 = _sc_sum(xs, srcp, dstp)
    cnt3 = _sc_cnt(dstc)
    return _encode_tc(
        sum3, sum3, cnt3, cnt3, x_dst,
        Wl[:, :DH].T, Wl[:, DH:].T, Wr.T,
        bl.reshape(1, D), gamma.reshape(1, D), beta.reshape(1, D))


def kernel(x_u, x_v, edge_index_adv, edge_index_dif,
           Wl_adv, bl_adv, Wr_adv, Wl_dif, bl_dif, Wr_dif,
           gamma, beta):
    h_adv = _relation(x_u, x_v, edge_index_adv, Wl_adv, bl_adv, Wr_adv,
                      gamma, beta)
    h_dif = _relation(x_v, x_u, edge_index_dif, Wl_dif, bl_dif, Wr_dif,
                      gamma, beta)
    return (h_adv, h_dif)
